# baseline (speedup 1.0000x reference)
import functools

import jax
import jax.numpy as jnp
from jax import lax
from jax.experimental import pallas as pl
from jax.experimental.pallas import tpu as pltpu
from jax.experimental.pallas import tpu_sc as plsc

N_NODES = 10000
N_EDGES = 320000
IN_CH = 128
OUT_CH = 32
HEADS = 4
H2 = 2 * HEADS

NC = 2
NS = 16
NW = NC * NS
LANES = 16

C_EDGES = 2560
N_CHUNKS = N_EDGES // C_EDGES
CHUNKS_PER_W = -(-N_CHUNKS // NW)

ROW_BLOCK = N_NODES


def _alpha_body(x_ref, w_ref, b_ref, attl_ref, attr_ref, out_ref,
                a_ref, c_ref):
    @pl.when(pl.program_id(0) == 0)
    def _():
        row = lax.broadcasted_iota(jnp.int32, (H2, IN_CH), 0)
        k = lax.broadcasted_iota(jnp.int32, (H2, IN_CH), 1)
        attl = jnp.reshape(attl_ref[...], (1, IN_CH))
        attr = jnp.reshape(attr_ref[...], (1, IN_CH))
        seg_l = (row < HEADS) & (k >= row * OUT_CH) & (k < (row + 1) * OUT_CH)
        rr = row - HEADS
        seg_r = (row >= HEADS) & (k >= rr * OUT_CH) & (k < (rr + 1) * OUT_CH)
        st = (jnp.where(seg_l, jnp.broadcast_to(attl, (H2, IN_CH)), 0.0)
              + jnp.where(seg_r, jnp.broadcast_to(attr, (H2, IN_CH)), 0.0))
        a_ref[...] = lax.dot_general(w_ref[...], st, (((0,), (1,)), ((), ())),
                                     preferred_element_type=jnp.float32,
                                     precision=lax.Precision.HIGHEST)
        c_ref[...] = lax.dot_general(st, b_ref[...], (((1,), (1,)), ((), ())),
                                     preferred_element_type=jnp.float32,
                                     precision=lax.Precision.HIGHEST)
    out_ref[...] = lax.dot_general(a_ref[...], x_ref[...],
                                   (((0,), (1,)), ((), ())),
                                   preferred_element_type=jnp.float32
                                   ) + c_ref[...]


_alpha_call = pl.pallas_call(
    _alpha_body,
    grid=(N_NODES // ROW_BLOCK,),
    in_specs=[
        pl.BlockSpec((ROW_BLOCK, IN_CH), lambda i: (i, 0)),
        pl.BlockSpec((IN_CH, IN_CH), lambda i: (0, 0)),
        pl.BlockSpec((1, IN_CH), lambda i: (0, 0)),
        pl.BlockSpec((1, HEADS, OUT_CH), lambda i: (0, 0, 0)),
        pl.BlockSpec((1, HEADS, OUT_CH), lambda i: (0, 0, 0)),
    ],
    out_specs=pl.BlockSpec((H2, ROW_BLOCK), lambda i: (0, i)),
    out_shape=jax.ShapeDtypeStruct((H2, N_NODES), jnp.float32),
    scratch_shapes=[
        pltpu.VMEM((IN_CH, H2), jnp.float32),
        pltpu.VMEM((H2, 1), jnp.float32),
    ],
)


def _edge_body(tab_hbm, ei_hbm, out_hbm, tab_v, ei_v, out_v,
               tab_sem, ei_sem0, ei_sem1, out_sem0, out_sem1):
    wid = lax.axis_index("s") * NC + lax.axis_index("c")
    ei_sems = (ei_sem0, ei_sem1)
    out_sems = (out_sem0, out_sem1)

    def ei_copies(b, cid):
        base_e = cid * C_EDGES
        eb = b * (2 * C_EDGES)
        return (
            pltpu.make_async_copy(ei_hbm.at[pl.ds(base_e, C_EDGES)],
                                  ei_v.at[pl.ds(eb, C_EDGES)], ei_sems[b]),
            pltpu.make_async_copy(ei_hbm.at[pl.ds(N_EDGES + base_e, C_EDGES)],
                                  ei_v.at[pl.ds(eb + C_EDGES, C_EDGES)],
                                  ei_sems[b]),
        )

    def out_copy(b, cid):
        return pltpu.make_async_copy(
            out_v.at[pl.ds(b * (C_EDGES * HEADS), C_EDGES * HEADS)],
            out_hbm.at[pl.ds(cid * C_EDGES * HEADS, C_EDGES * HEADS)],
            out_sems[b])

    tab_cp = pltpu.make_async_copy(tab_hbm, tab_v, tab_sem)
    tab_cp.start()
    for c in ei_copies(0, wid):
        c.start()

    for t in range(CHUNKS_PER_W):
        cid = t * NW + wid
        b = t % 2

        @pl.when(cid < N_CHUNKS)
        def _():
            for c in ei_copies(b, cid):
                c.wait()
            if t + 1 < CHUNKS_PER_W:
                ncid = (t + 1) * NW + wid

                @pl.when(ncid < N_CHUNKS)
                def _():
                    for c in ei_copies(1 - b, ncid):
                        c.start()
            if t == 0:
                tab_cp.wait()
            if t >= 2:
                out_copy(b, (t - 2) * NW + wid).wait()

            eb = b * (2 * C_EDGES)
            ob = b * (C_EDGES * HEADS)

            @plsc.parallel_loop(0, C_EDGES // LANES, unroll=4)
            def body(j):
                sv = ei_v[pl.ds(eb + j * LANES, LANES)]
                dv = ei_v[pl.ds(eb + C_EDGES + j * LANES, LANES)]
                obase = ob + (j // 8) * (HEADS * 128) + (j % 8) * LANES
                for hd in range(HEADS):
                    a = plsc.load_gather(tab_v, [sv + hd * N_NODES])
                    r = plsc.load_gather(tab_v, [dv + (HEADS + hd) * N_NODES])
                    v = a + r
                    res = jnp.where(v >= 0.0, v, v * jnp.float32(0.01))
                    out_v[pl.ds(obase + hd * 128, LANES)] = res

            out_copy(b, cid).start()

    for t in (CHUNKS_PER_W - 2, CHUNKS_PER_W - 1):
        cid = t * NW + wid

        @pl.when(cid < N_CHUNKS)
        def _():
            out_copy(t % 2, cid).wait()


@functools.cache
def _edge_kernel():
    return pl.kernel(
        _edge_body,
        mesh=plsc.VectorSubcoreMesh(core_axis_name="c", subcore_axis_name="s",
                                    num_cores=NC, num_subcores=NS),
        compiler_params=pltpu.CompilerParams(needs_layout_passes=False),
        out_type=jax.ShapeDtypeStruct((N_EDGES * HEADS,), jnp.float32),
        scratch_types=[
            pltpu.VMEM((N_NODES * H2,), jnp.float32),
            pltpu.VMEM((2 * 2 * C_EDGES,), jnp.int32),
            pltpu.VMEM((2 * C_EDGES * HEADS,), jnp.float32),
            pltpu.SemaphoreType.DMA,
            pltpu.SemaphoreType.DMA,
            pltpu.SemaphoreType.DMA,
            pltpu.SemaphoreType.DMA,
            pltpu.SemaphoreType.DMA,
        ],
    )


def kernel(x, edge_index, W, b, att_l, att_r):
    eif = edge_index.astype(jnp.int32).reshape(-1)
    alpha = _alpha_call(x, W, b.reshape(1, IN_CH), att_l, att_r)
    out_flat = _edge_kernel()(alpha.reshape(-1), eif)
    return (out_flat.reshape(N_EDGES // 128, HEADS, 128)
            .transpose(0, 2, 1).reshape(N_EDGES, HEADS))


# speedup vs baseline: 1.0054x; 1.0054x over previous
import functools

import jax
import jax.numpy as jnp
from jax import lax
from jax.experimental import pallas as pl
from jax.experimental.pallas import tpu as pltpu
from jax.experimental.pallas import tpu_sc as plsc

N_NODES = 10000
N_EDGES = 320000
IN_CH = 128
OUT_CH = 32
HEADS = 4
H2 = 2 * HEADS

NC = 2
NS = 16
NW = NC * NS
LANES = 16

C_EDGES = 2560
N_CHUNKS = N_EDGES // C_EDGES
CHUNKS_PER_W = -(-N_CHUNKS // NW)

ROW_BLOCK = N_NODES


def _alpha_body(x_ref, w_ref, b_ref, attl_ref, attr_ref, out_ref,
                a_ref, c_ref):
    @pl.when(pl.program_id(0) == 0)
    def _():
        row = lax.broadcasted_iota(jnp.int32, (H2, IN_CH), 0)
        k = lax.broadcasted_iota(jnp.int32, (H2, IN_CH), 1)
        attl = jnp.reshape(attl_ref[...], (1, IN_CH))
        attr = jnp.reshape(attr_ref[...], (1, IN_CH))
        seg_l = (row < HEADS) & (k >= row * OUT_CH) & (k < (row + 1) * OUT_CH)
        rr = row - HEADS
        seg_r = (row >= HEADS) & (k >= rr * OUT_CH) & (k < (rr + 1) * OUT_CH)
        st = (jnp.where(seg_l, jnp.broadcast_to(attl, (H2, IN_CH)), 0.0)
              + jnp.where(seg_r, jnp.broadcast_to(attr, (H2, IN_CH)), 0.0))
        a_ref[...] = lax.dot_general(w_ref[...], st, (((0,), (1,)), ((), ())),
                                     preferred_element_type=jnp.float32,
                                     precision=lax.Precision.HIGHEST)
        c_ref[...] = lax.dot_general(st, b_ref[...], (((1,), (1,)), ((), ())),
                                     preferred_element_type=jnp.float32,
                                     precision=lax.Precision.HIGHEST)
    out_ref[...] = lax.dot_general(a_ref[...], x_ref[...],
                                   (((0,), (1,)), ((), ())),
                                   preferred_element_type=jnp.float32
                                   ) + c_ref[...]


_alpha_call = pl.pallas_call(
    _alpha_body,
    grid=(N_NODES // ROW_BLOCK,),
    in_specs=[
        pl.BlockSpec((ROW_BLOCK, IN_CH), lambda i: (i, 0)),
        pl.BlockSpec((IN_CH, IN_CH), lambda i: (0, 0)),
        pl.BlockSpec((1, IN_CH), lambda i: (0, 0)),
        pl.BlockSpec((1, HEADS, OUT_CH), lambda i: (0, 0, 0)),
        pl.BlockSpec((1, HEADS, OUT_CH), lambda i: (0, 0, 0)),
    ],
    out_specs=pl.BlockSpec((H2, ROW_BLOCK), lambda i: (0, i)),
    out_shape=jax.ShapeDtypeStruct((H2, N_NODES), jnp.float32),
    scratch_shapes=[
        pltpu.VMEM((IN_CH, H2), jnp.float32),
        pltpu.VMEM((H2, 1), jnp.float32),
    ],
)


def _edge_body(tab_hbm, ei_hbm, out_hbm, tab_v, ei_v, out_v,
               tab_sem, ei_sem0, ei_sem1, out_sem0, out_sem1):
    wid = lax.axis_index("s") * NC + lax.axis_index("c")
    ei_sems = (ei_sem0, ei_sem1)
    out_sems = (out_sem0, out_sem1)

    def ei_copies(b, cid):
        base_e = cid * C_EDGES
        eb = b * (2 * C_EDGES)
        return (
            pltpu.make_async_copy(ei_hbm.at[pl.ds(base_e, C_EDGES)],
                                  ei_v.at[pl.ds(eb, C_EDGES)], ei_sems[b]),
            pltpu.make_async_copy(ei_hbm.at[pl.ds(N_EDGES + base_e, C_EDGES)],
                                  ei_v.at[pl.ds(eb + C_EDGES, C_EDGES)],
                                  ei_sems[b]),
        )

    def out_copy(b, cid):
        return pltpu.make_async_copy(
            out_v.at[pl.ds(b * (C_EDGES * HEADS), C_EDGES * HEADS)],
            out_hbm.at[pl.ds(cid * C_EDGES * HEADS, C_EDGES * HEADS)],
            out_sems[b])

    tab_cp = pltpu.make_async_copy(tab_hbm, tab_v, tab_sem)
    tab_cp.start()
    for c in ei_copies(0, wid):
        c.start()

    for t in range(CHUNKS_PER_W):
        cid = t * NW + wid
        b = t % 2

        @pl.when(cid < N_CHUNKS)
        def _():
            for c in ei_copies(b, cid):
                c.wait()
            if t + 1 < CHUNKS_PER_W:
                ncid = (t + 1) * NW + wid

                @pl.when(ncid < N_CHUNKS)
                def _():
                    for c in ei_copies(1 - b, ncid):
                        c.start()
            if t == 0:
                tab_cp.wait()
            if t >= 2:
                out_copy(b, (t - 2) * NW + wid).wait()

            eb = b * (2 * C_EDGES)
            ob = b * (C_EDGES * HEADS)

            @plsc.parallel_loop(0, C_EDGES // LANES, unroll=2)
            def body(j):
                sv = ei_v[pl.ds(eb + j * LANES, LANES)]
                dv = ei_v[pl.ds(eb + C_EDGES + j * LANES, LANES)]
                obase = ob + (j // 8) * (HEADS * 128) + (j % 8) * LANES
                for hd in range(HEADS):
                    a = plsc.load_gather(tab_v, [sv + hd * N_NODES])
                    r = plsc.load_gather(tab_v, [dv + (HEADS + hd) * N_NODES])
                    v = a + r
                    res = jnp.where(v >= 0.0, v, v * jnp.float32(0.01))
                    out_v[pl.ds(obase + hd * 128, LANES)] = res

            out_copy(b, cid).start()

    for t in (CHUNKS_PER_W - 2, CHUNKS_PER_W - 1):
        cid = t * NW + wid

        @pl.when(cid < N_CHUNKS)
        def _():
            out_copy(t % 2, cid).wait()


@functools.cache
def _edge_kernel():
    return pl.kernel(
        _edge_body,
        mesh=plsc.VectorSubcoreMesh(core_axis_name="c", subcore_axis_name="s",
                                    num_cores=NC, num_subcores=NS),
        compiler_params=pltpu.CompilerParams(needs_layout_passes=False),
        out_type=jax.ShapeDtypeStruct((N_EDGES * HEADS,), jnp.float32),
        scratch_types=[
            pltpu.VMEM((N_NODES * H2,), jnp.float32),
            pltpu.VMEM((2 * 2 * C_EDGES,), jnp.int32),
            pltpu.VMEM((2 * C_EDGES * HEADS,), jnp.float32),
            pltpu.SemaphoreType.DMA,
            pltpu.SemaphoreType.DMA,
            pltpu.SemaphoreType.DMA,
            pltpu.SemaphoreType.DMA,
            pltpu.SemaphoreType.DMA,
        ],
    )


def kernel(x, edge_index, W, b, att_l, att_r):
    eif = edge_index.astype(jnp.int32).reshape(-1)
    alpha = _alpha_call(x, W, b.reshape(1, IN_CH), att_l, att_r)
    out_flat = _edge_kernel()(alpha.reshape(-1), eif)
    return (out_flat.reshape(N_EDGES // 128, HEADS, 128)
            .transpose(0, 2, 1).reshape(N_EDGES, HEADS))
